# baseline (speedup 1.0000x reference)
import functools

import jax
import jax.numpy as jnp
from jax.experimental import pallas as pl
from jax.experimental.pallas import tpu as pltpu

MEM = 8192
BS = 256


def _body(h_full_ref, h_blk_ref, Wq_ref, bq_ref, Wk_ref, bk_ref,
          Wv_ref, bv_ref, Wor_ref, Woh_ref, bo_ref,
          out_ref, kn_s, v_s):
    b = pl.program_id(0)
    i = pl.program_id(1)

    @pl.when((b == 0) & (i == 0))
    def _init():
        hbar = jnp.mean(h_full_ref[:], axis=0)
        k = jnp.dot(hbar, Wk_ref[:], preferred_element_type=jnp.float32)
        k = k + bk_ref[0]
        n = jnp.sqrt(jnp.sum(k * k, axis=-1, keepdims=True))
        kn_s[:] = (k / jnp.maximum(n, 1e-12)).astype(jnp.bfloat16)
        v = jnp.dot(hbar, Wv_ref[:], preferred_element_type=jnp.float32)
        v_s[:] = (v + bv_ref[0]).astype(jnp.bfloat16)

    h = h_blk_ref[0]
    q = jnp.dot(h, Wq_ref[:], preferred_element_type=jnp.float32) + bq_ref[0]
    n = jnp.sqrt(jnp.sum(q * q, axis=-1, keepdims=True))
    qn = (q / jnp.maximum(n, 1e-12)).astype(jnp.bfloat16)
    sim = jax.lax.dot_general(qn, kn_s[:], (((1,), (1,)), ((), ())),
                              preferred_element_type=jnp.float32)
    e = jnp.exp(sim)
    denom = jnp.sum(e, axis=-1, keepdims=True)
    r = jnp.dot(e.astype(jnp.bfloat16), v_s[:],
                preferred_element_type=jnp.float32) / denom
    out = jnp.dot(r, Wor_ref[:], preferred_element_type=jnp.float32)
    out += jnp.dot(h, Woh_ref[:], preferred_element_type=jnp.float32)
    out_ref[0] = out + bo_ref[0]


@jax.jit
def kernel(hidden_states, Wq, bq, Wk, bk, Wv, bv, Wo, bo):
    B, S, H = hidden_states.shape
    K = Wq.shape[1]
    V = Wv.shape[1]
    assert MEM % S == 0 and B > 1
    nblk = S // BS

    Wor = Wo[:V]
    Woh = Wo[V:].astype(jnp.bfloat16)
    hidden_bf = hidden_states.astype(jnp.bfloat16)
    Wq_bf = Wq.astype(jnp.bfloat16)

    grid = (B, nblk)
    out = pl.pallas_call(
        _body,
        grid=grid,
        in_specs=[
            pl.BlockSpec((B, S, H), lambda b, i: (0, 0, 0)),
            pl.BlockSpec((1, BS, H), lambda b, i: (b, i, 0)),
            pl.BlockSpec((H, K), lambda b, i: (0, 0)),
            pl.BlockSpec((1, K), lambda b, i: (0, 0)),
            pl.BlockSpec((H, K), lambda b, i: (0, 0)),
            pl.BlockSpec((1, K), lambda b, i: (0, 0)),
            pl.BlockSpec((H, V), lambda b, i: (0, 0)),
            pl.BlockSpec((1, V), lambda b, i: (0, 0)),
            pl.BlockSpec((V, H), lambda b, i: (0, 0)),
            pl.BlockSpec((H, H), lambda b, i: (0, 0)),
            pl.BlockSpec((1, H), lambda b, i: (0, 0)),
        ],
        out_specs=pl.BlockSpec((1, BS, H), lambda b, i: (b, i, 0)),
        out_shape=jax.ShapeDtypeStruct((B, S, H), jnp.float32),
        scratch_shapes=[
            pltpu.VMEM((S, K), jnp.bfloat16),
            pltpu.VMEM((S, V), jnp.bfloat16),
        ],
    )(hidden_states, hidden_bf, Wq_bf, bq.reshape(1, K), Wk,
      bk.reshape(1, K), Wv, bv.reshape(1, V), Wor, Woh, bo.reshape(1, H))
    return out


# speedup vs baseline: 1.0260x; 1.0260x over previous
import functools

import jax
import jax.numpy as jnp
from jax.experimental import pallas as pl
from jax.experimental.pallas import tpu as pltpu

MEM = 8192
BS = 256
VPAD = 128


def _body(h_ref, Wq_ref, bq_ref, Wk_ref, bk_ref, Wv_ref, bv_ref,
          Wor_ref, Woh_ref, bo_ref, out_ref, h_cache, kn_s, v_s,
          *, nblk):
    g = pl.program_id(0)

    @pl.when(g < nblk)
    def _phase1():
        hf = h_ref[...]
        hbar = jnp.mean(hf, axis=0)
        hbar_bf = hbar.astype(jnp.bfloat16)
        k = jnp.dot(hbar_bf, Wk_ref[...],
                    preferred_element_type=jnp.float32) + bk_ref[0]
        n = jnp.sqrt(jnp.sum(k * k, axis=-1, keepdims=True))
        kn = k / jnp.maximum(n, 1e-12)
        v = jnp.dot(hbar_bf, Wv_ref[...],
                    preferred_element_type=jnp.float32) + bv_ref[0]
        rows = pl.ds(g * BS, BS)
        kn_s[rows, :] = kn.astype(jnp.bfloat16)
        vcols = v.shape[-1]
        pad = jnp.concatenate(
            [v, jnp.ones((BS, 1), jnp.float32),
             jnp.zeros((BS, VPAD - vcols - 1), jnp.float32)], axis=-1)
        v_s[rows, :] = pad.astype(jnp.bfloat16)
        h_cache[:, rows, :] = hf.astype(jnp.bfloat16)

    @pl.when(g >= nblk)
    def _phase2():
        t = g - nblk
        b = t // nblk
        i = t % nblk
        h = h_cache[b, pl.ds(i * BS, BS), :]
        q = jnp.dot(h, Wq_ref[...],
                    preferred_element_type=jnp.float32) + bq_ref[0]
        n = jnp.sqrt(jnp.sum(q * q, axis=-1, keepdims=True))
        qn = (q / jnp.maximum(n, 1e-12)).astype(jnp.bfloat16)
        sim = jax.lax.dot_general(qn, kn_s[...], (((1,), (1,)), ((), ())),
                                  preferred_element_type=jnp.float32)
        e = jnp.exp(sim).astype(jnp.bfloat16)
        rfull = jnp.dot(e, v_s[...], preferred_element_type=jnp.float32)
        denom = rfull[:, Wq_ref.shape[1]:Wq_ref.shape[1] + 1]
        out = jnp.dot(rfull.astype(jnp.bfloat16), Wor_ref[...],
                      preferred_element_type=jnp.float32) / denom
        out += jnp.dot(h, Woh_ref[...], preferred_element_type=jnp.float32)
        out_ref[0] = out + bo_ref[0]


@jax.jit
def kernel(hidden_states, Wq, bq, Wk, bk, Wv, bv, Wo, bo):
    B, S, H = hidden_states.shape
    K = Wq.shape[1]
    V = Wv.shape[1]
    assert MEM % S == 0 and B > 1 and S % BS == 0
    nblk = S // BS

    bf = jnp.bfloat16
    Wor = jnp.zeros((VPAD, H), jnp.float32).at[:V].set(Wo[:V]).astype(bf)
    Woh = Wo[V:].astype(bf)
    Wq_bf = Wq.astype(bf)
    Wk_bf = Wk.astype(bf)
    Wv_bf = Wv.astype(bf)

    grid = (nblk + B * nblk,)

    def h_map(g):
        return (0, jnp.minimum(g, nblk - 1), 0)

    def out_map(g):
        t = jnp.maximum(g - nblk, 0)
        return (t // nblk, t % nblk, 0)

    const = lambda g: (0, 0)

    out = pl.pallas_call(
        functools.partial(_body, nblk=nblk),
        grid=grid,
        in_specs=[
            pl.BlockSpec((B, BS, H), h_map),
            pl.BlockSpec((H, K), const),
            pl.BlockSpec((1, K), const),
            pl.BlockSpec((H, K), const),
            pl.BlockSpec((1, K), const),
            pl.BlockSpec((H, V), const),
            pl.BlockSpec((1, V), const),
            pl.BlockSpec((VPAD, H), const),
            pl.BlockSpec((H, H), const),
            pl.BlockSpec((1, H), const),
        ],
        out_specs=pl.BlockSpec((1, BS, H), out_map),
        out_shape=jax.ShapeDtypeStruct((B, S, H), jnp.float32),
        scratch_shapes=[
            pltpu.VMEM((B, S, H), bf),
            pltpu.VMEM((S, K), bf),
            pltpu.VMEM((S, VPAD), bf),
        ],
    )(hidden_states, Wq_bf, bq.reshape(1, K), Wk_bf, bk.reshape(1, K),
      Wv_bf, bv.reshape(1, V), Wor, Woh, bo.reshape(1, H))
    return out


# speedup vs baseline: 1.0353x; 1.0091x over previous
import functools

import jax
import jax.numpy as jnp
from jax.experimental import pallas as pl
from jax.experimental.pallas import tpu as pltpu

MEM = 8192
BS = 256
VPAD = 128


def _tables_body(h_ref, Wk_ref, bk_ref, Wv_ref, bv_ref,
                 hbf_ref, kn_ref, v_ref):
    hf = h_ref[...]
    hbf_ref[...] = hf.astype(jnp.bfloat16)
    hbar = jnp.mean(hf, axis=0).astype(jnp.bfloat16)
    k = jnp.dot(hbar, Wk_ref[...],
                preferred_element_type=jnp.float32) + bk_ref[0]
    n = jnp.sqrt(jnp.sum(k * k, axis=-1, keepdims=True))
    kn_ref[...] = (k / jnp.maximum(n, 1e-12)).astype(jnp.bfloat16)
    v = jnp.dot(hbar, Wv_ref[...],
                preferred_element_type=jnp.float32) + bv_ref[0]
    S = v.shape[0]
    vcols = v.shape[-1]
    pad = jnp.concatenate(
        [v, jnp.ones((S, 1), jnp.float32),
         jnp.zeros((S, VPAD - vcols - 1), jnp.float32)], axis=-1)
    v_ref[...] = pad.astype(jnp.bfloat16)


def _attn_body(h_ref, Wq_ref, bq_ref, kn_ref, v_ref,
               Wor_ref, Woh_ref, bo_ref, out_ref):
    h = h_ref[0]
    q = jnp.dot(h, Wq_ref[...],
                preferred_element_type=jnp.float32) + bq_ref[0]
    n = jnp.sqrt(jnp.sum(q * q, axis=-1, keepdims=True))
    qn = (q / jnp.maximum(n, 1e-12)).astype(jnp.bfloat16)
    sim = jax.lax.dot_general(qn, kn_ref[...], (((1,), (1,)), ((), ())),
                              preferred_element_type=jnp.float32)
    e = jnp.exp(sim).astype(jnp.bfloat16)
    rfull = jnp.dot(e, v_ref[...], preferred_element_type=jnp.float32)
    denom = rfull[:, Wq_ref.shape[1]:Wq_ref.shape[1] + 1]
    out = jnp.dot(rfull.astype(jnp.bfloat16), Wor_ref[...],
                  preferred_element_type=jnp.float32) / denom
    out += jnp.dot(h, Woh_ref[...], preferred_element_type=jnp.float32)
    out_ref[0] = out + bo_ref[0]


@jax.jit
def kernel(hidden_states, Wq, bq, Wk, bk, Wv, bv, Wo, bo):
    B, S, H = hidden_states.shape
    K = Wq.shape[1]
    V = Wv.shape[1]
    assert MEM % S == 0 and B > 1 and S % BS == 0
    nblk = S // BS

    bf = jnp.bfloat16
    Wor = jnp.zeros((VPAD, H), jnp.float32).at[:V].set(Wo[:V]).astype(bf)
    Woh = Wo[V:].astype(bf)
    Wq_bf = Wq.astype(bf)
    Wk_bf = Wk.astype(bf)
    Wv_bf = Wv.astype(bf)

    hbf, kn, vals = pl.pallas_call(
        _tables_body,
        out_shape=[jax.ShapeDtypeStruct((B, S, H), bf),
                   jax.ShapeDtypeStruct((S, K), bf),
                   jax.ShapeDtypeStruct((S, VPAD), bf)],
    )(hidden_states, Wk_bf, bk.reshape(1, K), Wv_bf, bv.reshape(1, V))

    const = lambda b, i: (0, 0)
    out = pl.pallas_call(
        _attn_body,
        grid=(B, nblk),
        in_specs=[
            pl.BlockSpec((1, BS, H), lambda b, i: (b, i, 0)),
            pl.BlockSpec((H, K), const),
            pl.BlockSpec((1, K), const),
            pl.BlockSpec((S, K), const),
            pl.BlockSpec((S, VPAD), const),
            pl.BlockSpec((VPAD, H), const),
            pl.BlockSpec((H, H), const),
            pl.BlockSpec((1, H), const),
        ],
        out_specs=pl.BlockSpec((1, BS, H), lambda b, i: (b, i, 0)),
        out_shape=jax.ShapeDtypeStruct((B, S, H), jnp.float32),
    )(hbf, Wq_bf, bq.reshape(1, K), kn, vals, Wor, Woh, bo.reshape(1, H))
    return out
